# Initial kernel scaffold; baseline (speedup 1.0000x reference)
import jax
import jax.numpy as jnp
from jax.experimental import pallas as pl


def kernel(x, edge_index, edge_weight, cat_list, Wr_real, br_real, Watt_real, batt_real, emb_real, Wr_plan, br_plan, Watt_plan, batt_plan, emb_plan, Wr_other, br_other, Watt_other, batt_other, emb_other, Wg, bg, Wa, ba):
    raise NotImplementedError("write your pallas kernel here")


# speedup vs baseline: 5.2106x; 5.2106x over previous
import functools

import jax
import jax.numpy as jnp
from jax import lax
from jax.experimental import pallas as pl
from jax.experimental.pallas import tpu as pltpu
from jax.experimental.pallas import tpu_sc as plsc

N = 10000
D = 128
E = 320000
NC = 2
NS = 16
NW = NC * NS
EPW = E // NW
CH = 80
NCHUNK = EPW // CH
RPT = 624


def _sc_edge_phase(x, src, dst, ew, cat, a, w2tab, embtab):
  mesh = plsc.VectorSubcoreMesh(core_axis_name="c", subcore_axis_name="s")

  @functools.partial(
      pl.kernel,
      mesh=mesh,
      compiler_params=pltpu.CompilerParams(needs_layout_passes=False),
      out_type=(
          jax.ShapeDtypeStruct((NC * N, D), jnp.float32),
          jax.ShapeDtypeStruct((NW * N,), jnp.float32),
      ),
      scratch_types=[
          pltpu.VMEM((N,), jnp.int32),
          pltpu.VMEM((N,), jnp.float32),
          pltpu.VMEM((30 * D,), jnp.float32),
          pltpu.VMEM((30 * D,), jnp.float32),
          pltpu.VMEM((N,), jnp.float32),
          pltpu.VMEM((CH,), jnp.int32),
          pltpu.VMEM((CH,), jnp.int32),
          pltpu.VMEM((CH,), jnp.int32),
          pltpu.VMEM((CH, D), jnp.float32),
          pltpu.VMEM((16, D), jnp.float32),
          pltpu.VMEM_SHARED((N, D), jnp.float32),
          pltpu.SemaphoreType.DMA,
      ],
  )
  def k(x_h, src_h, dst_h, ew_h, cat_h, a_h, w2_h, emb_h,
        m_out, den_out,
        cat_v, a_v, w2_v, emb_v, den_v, src_v, dst_v, ew_v, rows, zbuf,
        m_sh, sem):
    cid = lax.axis_index("c")
    sid = lax.axis_index("s")
    wid = sid * NC + cid

    pltpu.sync_copy(cat_h, cat_v)
    pltpu.sync_copy(a_h, a_v)
    pltpu.sync_copy(w2_h, w2_v)
    pltpu.sync_copy(emb_h, emb_v)

    zero16 = jnp.zeros((16,), jnp.float32)

    def _zden(i, carry):
      den_v[pl.ds(i * 16, 16)] = zero16
      return carry

    lax.fori_loop(0, N // 16, _zden, 0)

    for r in range(16):
      for c8 in range(8):
        zbuf[r, pl.ds(c8 * 16, 16)] = zero16
    for j in range(39):
      r0 = pl.multiple_of(sid * RPT + j * 16, 8)
      pltpu.sync_copy(zbuf, m_sh.at[pl.ds(r0, 16)])

    @pl.when(sid == NS - 1)
    def _zero_tail():
      r0 = pl.multiple_of(N - 16, 8)
      pltpu.sync_copy(zbuf, m_sh.at[pl.ds(r0, 16)])

    plsc.subcore_barrier()

    lanes = lax.iota(jnp.int32, 16)

    def chunk_body(t, carry):
      base = pl.multiple_of(wid * EPW + t * CH, 8)
      pltpu.sync_copy(src_h.at[pl.ds(base, CH)], src_v)
      pltpu.sync_copy(dst_h.at[pl.ds(base, CH)], dst_v)
      pltpu.sync_copy(ew_h.at[pl.ds(base, CH)], ew_v)
      pltpu.async_copy(x_h.at[src_v], rows, sem).wait()
      for g in range(CH // 16):
        e16 = g * 16 + lanes
        d16 = dst_v[pl.ds(g * 16, 16)]
        w16 = ew_v[pl.ds(g * 16, 16)]
        cv = plsc.load_gather(cat_v, [d16])
        av = plsc.load_gather(a_v, [d16])
        cwb = (cv * 10 + w16) * D

        def f1(f, acc):
          fv = jnp.zeros((16,), jnp.int32) + f
          xv = plsc.load_gather(rows, [e16, fv])
          wv = plsc.load_gather(w2_v, [cwb + fv])
          return acc + xv * wv

        sdot = lax.fori_loop(0, D, f1, jnp.zeros((16,), jnp.float32))
        logit = av + sdot
        logit = jnp.where(logit >= 0.0, logit, -0.1 * logit)
        exv = jnp.exp(logit)
        plsc.addupdate_scatter(den_v, [d16], exv)

        def f2(f, carry2):
          fv = jnp.zeros((16,), jnp.int32) + f
          xv = plsc.load_gather(rows, [e16, fv])
          ev = plsc.load_gather(emb_v, [cwb + fv])
          plsc.store_scatter(rows, [e16, fv], exv * xv * ev)
          return carry2

        lax.fori_loop(0, D, f2, 0)
      pltpu.sync_copy(rows, m_sh.at[dst_v], add=True)
      return carry

    lax.fori_loop(0, NCHUNK, chunk_body, 0)
    plsc.subcore_barrier()

    pltpu.sync_copy(den_v, den_out.at[pl.ds(pl.multiple_of(wid * N, 8), N)])
    for j in range(7):
      r0 = pl.multiple_of(sid * RPT + j * CH, 8)
      pltpu.sync_copy(m_sh.at[pl.ds(r0, CH)], rows)
      pltpu.sync_copy(rows, m_out.at[pl.ds(pl.multiple_of(cid * N + r0, 8), CH)])
    r0 = pl.multiple_of(sid * RPT + 7 * CH, 8)
    pltpu.sync_copy(m_sh.at[pl.ds(r0, 64)], rows.at[pl.ds(0, 64)])
    pltpu.sync_copy(rows.at[pl.ds(0, 64)],
                    m_out.at[pl.ds(pl.multiple_of(cid * N + r0, 8), 64)])

    @pl.when(sid == NS - 1)
    def _copy_tail():
      r1 = pl.multiple_of(N - 16, 8)
      pltpu.sync_copy(m_sh.at[pl.ds(r1, 16)], rows.at[pl.ds(0, 16)])
      pltpu.sync_copy(rows.at[pl.ds(0, 16)],
                      m_out.at[pl.ds(pl.multiple_of(cid * N + r1, 8), 16)])

  return k(x, src, dst, ew, cat, a, w2tab, embtab)


_BN = 2000


def _tc_pre(x, w1, battrow, catc):
  def body(x_r, w1_r, bt_r, cat_r, o_r):
    av = jnp.dot(x_r[...], w1_r[...], preferred_element_type=jnp.float32)
    av = av + bt_r[...]
    c = cat_r[...]
    m0 = (c == 0).astype(jnp.float32)
    m1 = (c == 1).astype(jnp.float32)
    m2 = (c == 2).astype(jnp.float32)
    o_r[...] = m0 * av[:, 0:1] + m1 * av[:, 1:2] + m2 * av[:, 2:3]

  return pl.pallas_call(
      body,
      grid=(N // _BN,),
      in_specs=[
          pl.BlockSpec((_BN, D), lambda i: (i, 0)),
          pl.BlockSpec((D, 8), lambda i: (0, 0)),
          pl.BlockSpec((1, 8), lambda i: (0, 0)),
          pl.BlockSpec((_BN, 1), lambda i: (i, 0)),
      ],
      out_specs=pl.BlockSpec((_BN, 1), lambda i: (i, 0)),
      out_shape=jax.ShapeDtypeStruct((N, 1), jnp.float32),
  )(x, w1, battrow, catc)


def _tc_post(x, m2, den, catc, wr3, br3, wg, bgrow, wa, barow):
  def body(x_r, m_r, den_r, cat_r, wr_r, br_r, wg_r, bg_r, wa_r, ba_r, o_r):
    msum = m_r[0] + m_r[1]
    den_b = den_r[...]
    inv = 1.0 / (den_b + 1e-16)
    mbar = msum * inv
    sfrac = den_b * inv
    c = cat_r[...]
    m0 = (c == 0).astype(jnp.float32)
    m1 = (c == 1).astype(jnp.float32)
    m2b = (c == 2).astype(jnp.float32)
    wr = wr_r[...]
    v = (m0 * jnp.dot(mbar, wr[0:D], preferred_element_type=jnp.float32)
         + m1 * jnp.dot(mbar, wr[D:2 * D], preferred_element_type=jnp.float32)
         + m2b * jnp.dot(mbar, wr[2 * D:3 * D],
                         preferred_element_type=jnp.float32))
    br = br_r[...]
    v = v + sfrac * (m0 * br[0:1] + m1 * br[1:2] + m2b * br[2:3])
    wg_b = wg_r[...]
    bg_b = bg_r[...]
    ga = jax.nn.sigmoid(
        jnp.dot(v, wg_b[0:D], preferred_element_type=jnp.float32) + bg_b)
    gb = jax.nn.sigmoid(
        jnp.dot(v, wg_b[D:2 * D], preferred_element_type=jnp.float32) + bg_b)
    fl = v - ga * v
    fr = gb * v
    out1 = m0 * fl + m1 * fr
    out2 = m0 * fl + m2b * fr
    out3 = m1 * fl + m2b * fr
    wa_b = wa_r[...]
    o_r[...] = jnp.tanh(
        jnp.dot(x_r[...], wa_b[0:D], preferred_element_type=jnp.float32)
        + jnp.dot(out1, wa_b[D:2 * D], preferred_element_type=jnp.float32)
        + jnp.dot(out2, wa_b[2 * D:3 * D], preferred_element_type=jnp.float32)
        + jnp.dot(out3, wa_b[3 * D:4 * D], preferred_element_type=jnp.float32)
        + ba_r[...])

  return pl.pallas_call(
      body,
      grid=(N // _BN,),
      in_specs=[
          pl.BlockSpec((_BN, D), lambda i: (i, 0)),
          pl.BlockSpec((NC, _BN, D), lambda i: (0, i, 0)),
          pl.BlockSpec((_BN, 1), lambda i: (i, 0)),
          pl.BlockSpec((_BN, 1), lambda i: (i, 0)),
          pl.BlockSpec((3 * D, D), lambda i: (0, 0)),
          pl.BlockSpec((8, D), lambda i: (0, 0)),
          pl.BlockSpec((2 * D, D), lambda i: (0, 0)),
          pl.BlockSpec((1, D), lambda i: (0, 0)),
          pl.BlockSpec((4 * D, D), lambda i: (0, 0)),
          pl.BlockSpec((1, D), lambda i: (0, 0)),
      ],
      out_specs=pl.BlockSpec((_BN, D), lambda i: (i, 0)),
      out_shape=jax.ShapeDtypeStruct((N, D), jnp.float32),
  )(x, m2, den, catc, wr3, br3, wg, bgrow, wa, barow)


def kernel(x, edge_index, edge_weight, cat_list,
           Wr_real, br_real, Watt_real, batt_real, emb_real,
           Wr_plan, br_plan, Watt_plan, batt_plan, emb_plan,
           Wr_other, br_other, Watt_other, batt_other, emb_other,
           Wg, bg, Wa, ba):
  src = edge_index[0]
  dst = edge_index[1]
  catc = cat_list[:, None]

  w1 = jnp.concatenate(
      [Watt_real[:D], Watt_plan[:D], Watt_other[:D],
       jnp.zeros((D, 5), jnp.float32)], axis=1)
  battrow = jnp.concatenate(
      [batt_real, batt_plan, batt_other, jnp.zeros((5,), jnp.float32)])[None]
  a = _tc_pre(x, w1, battrow, catc)[:, 0]

  w2s = jnp.stack([Watt_real[D:, 0], Watt_plan[D:, 0], Watt_other[D:, 0]], 0)
  embtab = jnp.stack([emb_real, emb_plan, emb_other], 0)
  w2tab = (embtab * w2s[:, None, :]).reshape(30 * D)
  embflat = embtab.reshape(30 * D)

  m_flat, den_flat = _sc_edge_phase(
      x, src, dst, edge_weight, cat_list, a, w2tab, embflat)
  m2 = m_flat.reshape(NC, N, D)
  den = den_flat.reshape(NW, N).sum(0)[:, None]

  wr3 = jnp.concatenate([Wr_real, Wr_plan, Wr_other], 0)
  br3 = jnp.concatenate(
      [br_real[None], br_plan[None], br_other[None],
       jnp.zeros((5, D), jnp.float32)], 0)

  return _tc_post(x, m2, den, catc, wr3, br3, Wg, bg[None], Wa, ba[None])


# speedup vs baseline: 5.3738x; 1.0313x over previous
import functools

import jax
import jax.numpy as jnp
from jax import lax
from jax.experimental import pallas as pl
from jax.experimental.pallas import tpu as pltpu
from jax.experimental.pallas import tpu_sc as plsc

N = 10000
D = 128
E = 320000
NC = 2
NS = 16
NW = NC * NS
EPW = E // NW
CH = 80
NCHUNK = EPW // CH
RPT = 624


def _sc_edge_phase(x, src, dst, ew, acat, w2tab, embtab):
  mesh = plsc.VectorSubcoreMesh(core_axis_name="c", subcore_axis_name="s")

  @functools.partial(
      pl.kernel,
      mesh=mesh,
      compiler_params=pltpu.CompilerParams(needs_layout_passes=False),
      out_type=(
          jax.ShapeDtypeStruct((NC * N, D), jnp.float32),
          jax.ShapeDtypeStruct((NW * N,), jnp.float32),
      ),
      scratch_types=[
          pltpu.VMEM((N,), jnp.int32),
          pltpu.VMEM((30 * D,), jnp.float32),
          pltpu.VMEM((30 * D,), jnp.float32),
          pltpu.VMEM((N,), jnp.float32),
          pltpu.VMEM((CH, D), jnp.float32),
          pltpu.VMEM((CH, D), jnp.float32),
          pltpu.VMEM((8, D), jnp.float32),
          pltpu.VMEM((CH,), jnp.int32),
          pltpu.VMEM((CH,), jnp.int32),
          pltpu.VMEM((CH,), jnp.int32),
          pltpu.VMEM((CH,), jnp.int32),
          pltpu.VMEM_SHARED((N, D), jnp.float32),
          pltpu.SemaphoreType.DMA,
          pltpu.SemaphoreType.DMA,
      ],
  )
  def k(x_h, src_h, dst_h, ew_h, acat_h, w2_h, emb_h,
        m_out, den_out,
        acat_v, w2_v, emb_v, den_v, rows0, rows1,
        zbuf, gidx0, gidx1, sidx0, ew_v, m_sh, g0, g1):
    cid = lax.axis_index("c")
    sid = lax.axis_index("s")
    wid = sid * NC + cid

    pltpu.sync_copy(acat_h, acat_v)
    pltpu.sync_copy(w2_h, w2_v)
    pltpu.sync_copy(emb_h, emb_v)

    zero16 = jnp.zeros((16,), jnp.float32)

    def _zden(i, carry):
      den_v[pl.ds(i * 16, 16)] = zero16
      return carry

    lax.fori_loop(0, N // 16, _zden, 0)

    for r in range(8):
      for c8 in range(8):
        zbuf[r, pl.ds(c8 * 16, 16)] = zero16
    for j in range(78):
      r0 = pl.multiple_of(sid * RPT + j * 8, 8)
      pltpu.sync_copy(zbuf, m_sh.at[pl.ds(r0, 8)])

    @pl.when(sid == NS - 1)
    def _zero_tail():
      r0 = pl.multiple_of(N - 16, 8)
      pltpu.sync_copy(zbuf, m_sh.at[pl.ds(r0, 8)])
      r1 = pl.multiple_of(N - 8, 8)
      pltpu.sync_copy(zbuf, m_sh.at[pl.ds(r1, 8)])

    plsc.subcore_barrier()

    lanes = lax.iota(jnp.int32, 16)
    rows_b = (rows0, rows1)
    gsem = (g0, g1)
    gidx = (gidx0, gidx1)

    def issue_gather(t, b):
      base = pl.multiple_of(wid * EPW + t * CH, 8)
      pltpu.sync_copy(src_h.at[pl.ds(base, CH)], gidx[b])
      pltpu.async_copy(x_h.at[gidx[b]], rows_b[b], gsem[b])

    def wait_gather(b):
      pltpu.make_async_copy(x_h.at[gidx[b]], rows_b[b], gsem[b]).wait()

    def load_chunk_meta(t):
      base = pl.multiple_of(wid * EPW + t * CH, 8)
      pltpu.sync_copy(dst_h.at[pl.ds(base, CH)], sidx0)
      pltpu.sync_copy(ew_h.at[pl.ds(base, CH)], ew_v)

    def compute(t, b):
      rows = rows_b[b]
      for g in range(CH // 16):
        e16 = g * 16 + lanes
        d16 = sidx0[pl.ds(g * 16, 16)]
        w16 = ew_v[pl.ds(g * 16, 16)]
        pk = plsc.load_gather(acat_v, [d16])
        cv = pk & 3
        av = plsc.bitcast(pk & ~3, jnp.float32)
        cwb = (cv * 10 + w16) * D

        def f1(j, acc2):
          a0, a1 = acc2
          f = j * 4
          for u in range(4):
            fv = jnp.zeros((16,), jnp.int32) + (f + u)
            xv = plsc.load_gather(rows, [e16, fv])
            wv = plsc.load_gather(w2_v, [cwb + fv])
            if u % 2 == 0:
              a0 = a0 + xv * wv
            else:
              a1 = a1 + xv * wv
          return a0, a1

        z16 = jnp.zeros((16,), jnp.float32)
        sa0, sa1 = lax.fori_loop(0, D // 4, f1, (z16, z16))
        logit = av + sa0 + sa1
        logit = jnp.where(logit >= 0.0, logit, -0.1 * logit)
        exv = jnp.exp(logit)
        plsc.addupdate_scatter(den_v, [d16], exv)

        def f2(j, carry2):
          f = j * 4
          for u in range(4):
            fv = jnp.zeros((16,), jnp.int32) + (f + u)
            xv = plsc.load_gather(rows, [e16, fv])
            ev = plsc.load_gather(emb_v, [cwb + fv])
            plsc.store_scatter(rows, [e16, fv], exv * xv * ev)
          return carry2

        lax.fori_loop(0, D // 4, f2, 0)

    issue_gather(0, 0)

    def pair_body(q, carry):
      t0 = q * 2
      wait_gather(0)
      issue_gather(t0 + 1, 1)
      load_chunk_meta(t0)
      compute(t0, 0)
      pltpu.sync_copy(rows0, m_sh.at[sidx0], add=True)
      wait_gather(1)
      issue_gather(t0 + 2, 0)
      load_chunk_meta(t0 + 1)
      compute(t0 + 1, 1)
      pltpu.sync_copy(rows1, m_sh.at[sidx0], add=True)
      return carry

    lax.fori_loop(0, (NCHUNK - 1) // 2, pair_body, 0)
    wait_gather(0)
    load_chunk_meta(NCHUNK - 1)
    compute(NCHUNK - 1, 0)
    pltpu.sync_copy(rows0, m_sh.at[sidx0], add=True)
    plsc.subcore_barrier()

    pltpu.sync_copy(den_v, den_out.at[pl.ds(pl.multiple_of(wid * N, 8), N)])
    for j in range(7):
      r0 = pl.multiple_of(sid * RPT + j * CH, 8)
      pltpu.sync_copy(m_sh.at[pl.ds(r0, CH)], rows0)
      pltpu.sync_copy(rows0,
                      m_out.at[pl.ds(pl.multiple_of(cid * N + r0, 8), CH)])
    r0 = pl.multiple_of(sid * RPT + 7 * CH, 8)
    pltpu.sync_copy(m_sh.at[pl.ds(r0, 64)], rows0.at[pl.ds(0, 64)])
    pltpu.sync_copy(rows0.at[pl.ds(0, 64)],
                    m_out.at[pl.ds(pl.multiple_of(cid * N + r0, 8), 64)])

    @pl.when(sid == NS - 1)
    def _copy_tail():
      r1 = pl.multiple_of(N - 16, 8)
      pltpu.sync_copy(m_sh.at[pl.ds(r1, 16)], rows0.at[pl.ds(0, 16)])
      pltpu.sync_copy(rows0.at[pl.ds(0, 16)],
                      m_out.at[pl.ds(pl.multiple_of(cid * N + r1, 8), 16)])

  return k(x, src, dst, ew, acat, w2tab, embtab)


_BN = 2000


def _tc_pre(x, w1, battrow, catc):
  def body(x_r, w1_r, bt_r, cat_r, o_r):
    av = jnp.dot(x_r[...], w1_r[...], preferred_element_type=jnp.float32)
    av = av + bt_r[...]
    c = cat_r[...]
    m0 = (c == 0).astype(jnp.float32)
    m1 = (c == 1).astype(jnp.float32)
    m2 = (c == 2).astype(jnp.float32)
    o_r[...] = m0 * av[:, 0:1] + m1 * av[:, 1:2] + m2 * av[:, 2:3]

  return pl.pallas_call(
      body,
      grid=(N // _BN,),
      in_specs=[
          pl.BlockSpec((_BN, D), lambda i: (i, 0)),
          pl.BlockSpec((D, 8), lambda i: (0, 0)),
          pl.BlockSpec((1, 8), lambda i: (0, 0)),
          pl.BlockSpec((_BN, 1), lambda i: (i, 0)),
      ],
      out_specs=pl.BlockSpec((_BN, 1), lambda i: (i, 0)),
      out_shape=jax.ShapeDtypeStruct((N, 1), jnp.float32),
  )(x, w1, battrow, catc)


def _tc_post(x, m2, den, catc, wr3, br3, wg, bgrow, wa, barow):
  def body(x_r, m_r, den_r, cat_r, wr_r, br_r, wg_r, bg_r, wa_r, ba_r, o_r):
    msum = m_r[0] + m_r[1]
    den_b = den_r[...]
    inv = 1.0 / (den_b + 1e-16)
    mbar = msum * inv
    sfrac = den_b * inv
    c = cat_r[...]
    m0 = (c == 0).astype(jnp.float32)
    m1 = (c == 1).astype(jnp.float32)
    m2b = (c == 2).astype(jnp.float32)
    wr = wr_r[...]
    v = (m0 * jnp.dot(mbar, wr[0:D], preferred_element_type=jnp.float32)
         + m1 * jnp.dot(mbar, wr[D:2 * D], preferred_element_type=jnp.float32)
         + m2b * jnp.dot(mbar, wr[2 * D:3 * D],
                         preferred_element_type=jnp.float32))
    br = br_r[...]
    v = v + sfrac * (m0 * br[0:1] + m1 * br[1:2] + m2b * br[2:3])
    wg_b = wg_r[...]
    bg_b = bg_r[...]
    ga = jax.nn.sigmoid(
        jnp.dot(v, wg_b[0:D], preferred_element_type=jnp.float32) + bg_b)
    gb = jax.nn.sigmoid(
        jnp.dot(v, wg_b[D:2 * D], preferred_element_type=jnp.float32) + bg_b)
    fl = v - ga * v
    fr = gb * v
    out1 = m0 * fl + m1 * fr
    out2 = m0 * fl + m2b * fr
    out3 = m1 * fl + m2b * fr
    wa_b = wa_r[...]
    o_r[...] = jnp.tanh(
        jnp.dot(x_r[...], wa_b[0:D], preferred_element_type=jnp.float32)
        + jnp.dot(out1, wa_b[D:2 * D], preferred_element_type=jnp.float32)
        + jnp.dot(out2, wa_b[2 * D:3 * D], preferred_element_type=jnp.float32)
        + jnp.dot(out3, wa_b[3 * D:4 * D], preferred_element_type=jnp.float32)
        + ba_r[...])

  return pl.pallas_call(
      body,
      grid=(N // _BN,),
      in_specs=[
          pl.BlockSpec((_BN, D), lambda i: (i, 0)),
          pl.BlockSpec((NC, _BN, D), lambda i: (0, i, 0)),
          pl.BlockSpec((_BN, 1), lambda i: (i, 0)),
          pl.BlockSpec((_BN, 1), lambda i: (i, 0)),
          pl.BlockSpec((3 * D, D), lambda i: (0, 0)),
          pl.BlockSpec((8, D), lambda i: (0, 0)),
          pl.BlockSpec((2 * D, D), lambda i: (0, 0)),
          pl.BlockSpec((1, D), lambda i: (0, 0)),
          pl.BlockSpec((4 * D, D), lambda i: (0, 0)),
          pl.BlockSpec((1, D), lambda i: (0, 0)),
      ],
      out_specs=pl.BlockSpec((_BN, D), lambda i: (i, 0)),
      out_shape=jax.ShapeDtypeStruct((N, D), jnp.float32),
  )(x, m2, den, catc, wr3, br3, wg, bgrow, wa, barow)


def kernel(x, edge_index, edge_weight, cat_list,
           Wr_real, br_real, Watt_real, batt_real, emb_real,
           Wr_plan, br_plan, Watt_plan, batt_plan, emb_plan,
           Wr_other, br_other, Watt_other, batt_other, emb_other,
           Wg, bg, Wa, ba):
  src = edge_index[0]
  dst = edge_index[1]
  catc = cat_list[:, None]

  w1 = jnp.concatenate(
      [Watt_real[:D], Watt_plan[:D], Watt_other[:D],
       jnp.zeros((D, 5), jnp.float32)], axis=1)
  battrow = jnp.concatenate(
      [batt_real, batt_plan, batt_other, jnp.zeros((5,), jnp.float32)])[None]
  a = _tc_pre(x, w1, battrow, catc)[:, 0]

  w2s = jnp.stack([Watt_real[D:, 0], Watt_plan[D:, 0], Watt_other[D:, 0]], 0)
  embtab = jnp.stack([emb_real, emb_plan, emb_other], 0)
  w2tab = (embtab * w2s[:, None, :]).reshape(30 * D)
  embflat = embtab.reshape(30 * D)

  acat = (jax.lax.bitcast_convert_type(a, jnp.int32) & ~3) | cat_list
  m_flat, den_flat = _sc_edge_phase(
      x, src, dst, edge_weight, acat, w2tab, embflat)
  m2 = m_flat.reshape(NC, N, D)
  den = den_flat.reshape(NW, N).sum(0)[:, None]

  wr3 = jnp.concatenate([Wr_real, Wr_plan, Wr_other], 0)
  br3 = jnp.concatenate(
      [br_real[None], br_plan[None], br_other[None],
       jnp.zeros((5, D), jnp.float32)], 0)

  return _tc_post(x, m2, den, catc, wr3, br3, Wg, bg[None], Wa, ba[None])
